# baseline (speedup 1.0000x reference)
import functools

import jax
import jax.numpy as jnp
from jax import lax
from jax.experimental import pallas as pl
from jax.experimental.pallas import tpu as pltpu
from jax.experimental.pallas import tpu_sc as plsc

N = 4096
EMB = 128
HID = 128
OUT = 64
E = 1000000

NC = 2
NS = 16
NW = NC * NS
L = 16


_RPW = N // NW


@functools.cache
def _make_sc_prologue():
    @functools.partial(
        pl.kernel,
        mesh=plsc.VectorSubcoreMesh(core_axis_name="c", subcore_axis_name="s"),
        out_type=[
            jax.ShapeDtypeStruct((N, EMB), jnp.float32),
            jax.ShapeDtypeStruct((_EPAD,), jnp.int32),
        ],
        scratch_types=[
            pltpu.VMEM((_RPW,), jnp.int32),
            pltpu.VMEM((_RPW, EMB), jnp.float32),
            pltpu.VMEM((_EMX,), jnp.int32),
            pltpu.VMEM((_EMX,), jnp.int32),
            pltpu.SemaphoreType.DMA,
        ],
    )
    def _sc_prologue(emb_hbm, idx_hbm, src_hbm, dst_hbm, x_hbm, flat_hbm,
                     idx_v, rows_v, src_v, dst_v, sem):
        c = lax.axis_index("c")
        s = lax.axis_index("s")
        wid = s * NC + c
        rbase = wid * _RPW
        pltpu.sync_copy(idx_hbm.at[pl.ds(rbase, _RPW)], idx_v)
        pltpu.async_copy(emb_hbm.at[idx_v], rows_v, sem).wait()
        pltpu.sync_copy(rows_v, x_hbm.at[pl.ds(rbase, _RPW)])

        def prep(base, ew):
            pltpu.sync_copy(src_hbm.at[pl.ds(base, ew)], src_v.at[pl.ds(0, ew)])
            pltpu.sync_copy(dst_hbm.at[pl.ds(base, ew)], dst_v.at[pl.ds(0, ew)])

            def idx_body(g, carry):
                for l in range(8):
                    o = g * (8 * L) + l * L
                    sidx = src_v[pl.ds(o, L)]
                    t = dst_v[pl.ds(o, L)]
                    src_v[pl.ds(o, L)] = (
                        lax.shift_left(lax.shift_right_logical(t, 7), 19)
                        + lax.shift_left(sidx, 7)
                        + lax.bitwise_and(t, 127)
                    )
                return carry

            lax.fori_loop(0, ew // (8 * L), idx_body, 0)
            pltpu.sync_copy(src_v.at[pl.ds(0, ew)], flat_hbm.at[pl.ds(base, ew)])

        @pl.when(c == 0)
        def _():
            prep(s * _E0, _E0)

        @pl.when(c == 1)
        def _():
            prep(NS * _E0 + s * _E1, _E1)

    return _sc_prologue


_BI = 512
_BJ = 512


def _dnorm_body(a_ref, o_ref, ar_ref, acc_ref):
    i = pl.program_id(1)

    @pl.when(i == 0)
    def _():
        acc_ref[...] = jnp.zeros_like(acc_ref)

    a = jnp.maximum(a_ref[...], 0.0)
    ar_ref[...] = a.astype(jnp.bfloat16)
    acc_ref[...] += jnp.sum(a, axis=0, keepdims=True)

    @pl.when(i == pl.num_programs(1) - 1)
    def _():
        deg = acc_ref[...]
        o_ref[...] = jnp.where(deg > 0.0, lax.rsqrt(deg), 0.0)


def _dnorm(A):
    return pl.pallas_call(
        _dnorm_body,
        grid=(N // _BJ, N // _BI),
        in_specs=[pl.BlockSpec((_BI, _BJ), lambda j, i: (i, j))],
        out_specs=[
            pl.BlockSpec((1, _BJ), lambda j, i: (0, j)),
            pl.BlockSpec((_BI, _BJ), lambda j, i: (i, j)),
        ],
        out_shape=[
            jax.ShapeDtypeStruct((1, N), jnp.float32),
            jax.ShapeDtypeStruct((N, N), jnp.bfloat16),
        ],
        scratch_shapes=[pltpu.VMEM((1, _BJ), jnp.float32)],
    )(A)


def _layer_body(a_ref, x_ref, drow_ref, dcol_ref, w_ref, b_ref, o_ref,
                acc_ref, xls_ref, *, relu_out, bi):
    j = pl.program_id(0)
    i = pl.program_id(1)

    @pl.when(j == 0)
    def _():
        xw = jnp.dot(x_ref[pl.ds(i * bi, bi), :], w_ref[...],
                     preferred_element_type=jnp.float32)
        xls_ref[pl.ds(i * bi, bi), :] = (
            drow_ref[pl.ds(i * bi, bi), :] * xw).astype(jnp.bfloat16)

    @pl.when(i == 0)
    def _():
        acc_ref[...] = jnp.zeros_like(acc_ref)

    acc_ref[...] += lax.dot_general(
        a_ref[...], xls_ref[pl.ds(i * bi, bi), :],
        (((0,), (0,)), ((), ())),
        preferred_element_type=jnp.float32,
    )

    @pl.when(i == pl.num_programs(1) - 1)
    def _():
        y = dcol_ref[...] * acc_ref[...] + b_ref[...]
        o_ref[...] = jnp.maximum(y, 0.0) if relu_out else y


def _layer(Ar, x, d_col, W, b, relu_out):
    din = x.shape[1]
    dout = W.shape[1]
    body = functools.partial(_layer_body, relu_out=relu_out, bi=_BI)
    return pl.pallas_call(
        body,
        grid=(N // _BJ, N // _BI),
        in_specs=[
            pl.BlockSpec((_BI, _BJ), lambda j, i: (i, j)),
            pl.BlockSpec((N, din), lambda j, i: (0, 0)),
            pl.BlockSpec((N, 1), lambda j, i: (0, 0)),
            pl.BlockSpec((_BJ, 1), lambda j, i: (j, 0)),
            pl.BlockSpec((din, dout), lambda j, i: (0, 0)),
            pl.BlockSpec((1, dout), lambda j, i: (0, 0)),
        ],
        out_specs=pl.BlockSpec((_BJ, dout), lambda j, i: (j, 0)),
        out_shape=jax.ShapeDtypeStruct((N, dout), jnp.float32),
        scratch_shapes=[
            pltpu.VMEM((_BJ, dout), jnp.float32),
            pltpu.VMEM((N, dout), jnp.bfloat16),
        ],
    )(Ar, x, d_col, d_col, W, b)


_BGC = 512


def _gram_body(zi_ref, zc_ref, o_ref):
    res = lax.dot_general(
        zi_ref[...], zc_ref[...],
        (((1,), (1,)), ((), ())),
        preferred_element_type=jnp.float32,
    )
    for t in range(_BGC // 128):
        o_ref[t, :, :] = res[:, t * 128:(t + 1) * 128]


def _gram(z):
    return pl.pallas_call(
        _gram_body,
        grid=(N // _BGC,),
        in_specs=[
            pl.BlockSpec((N, OUT), lambda c: (0, 0)),
            pl.BlockSpec((_BGC, OUT), lambda c: (c, 0)),
        ],
        out_specs=pl.BlockSpec((_BGC // 128, N, 128), lambda c: (c, 0, 0)),
        out_shape=jax.ShapeDtypeStruct((N // 128, N, 128), jnp.float32),
    )(z, z)


_CH = 128
_KF = 8
_N0 = 328
_N1 = 168
_E0 = _N0 * _CH
_E1 = _N1 * _CH
_EMX = max(_E0, _E1)
_EPAD = NS * (_E0 + _E1)


@functools.cache
def _make_sc_decode():
    @functools.partial(
        pl.kernel,
        mesh=plsc.VectorSubcoreMesh(core_axis_name="c", subcore_axis_name="s"),
        out_type=jax.ShapeDtypeStruct((_EPAD,), jnp.float32),
        scratch_types=[
            pltpu.VMEM((_EMX,), jnp.int32),
            pltpu.VMEM((_EMX,), jnp.float32),
            pltpu.SemaphoreType.DMA,
        ],
    )
    def _sc_decode(g_hbm, flat_hbm, out_hbm, flat_v, res_v, sem):
        c = lax.axis_index("c")
        s = lax.axis_index("s")

        def run(base, nchunks):
            ew = nchunks * _CH
            pltpu.sync_copy(flat_hbm.at[pl.ds(base, ew)], flat_v.at[pl.ds(0, ew)])

            def group(g, carry):
                copies = []
                for j in range(_KF):
                    off = (g * _KF + j) * _CH
                    copies.append(
                        pltpu.async_copy(g_hbm.at[flat_v.at[pl.ds(off, _CH)]],
                                         res_v.at[pl.ds(off, _CH)], sem))
                for cp in copies:
                    cp.wait()
                return carry

            lax.fori_loop(0, nchunks // _KF, group, 0)
            pltpu.sync_copy(res_v.at[pl.ds(0, ew)], out_hbm.at[pl.ds(base, ew)])

        @pl.when(c == 0)
        def _():
            run(s * _E0, _N0)

        @pl.when(c == 1)
        def _():
            run(NS * _E0 + s * _E1, _N1)

    return _sc_decode


def kernel(node_indices, attn_matrix, edge_label_index, emb_table, W1, b1, W2, b2):
    idx32 = node_indices.astype(jnp.int32)
    eli = edge_label_index.astype(jnp.int32)
    pad = jnp.zeros((2, _EPAD - E), jnp.int32)
    eli_p = jnp.concatenate([eli, pad], axis=1)
    x, flat = _make_sc_prologue()(emb_table, idx32, eli_p[0], eli_p[1])

    d_row, Ar = _dnorm(attn_matrix)
    d_col = d_row.reshape(N, 1)

    h = _layer(Ar, x, d_col, W1, b1.reshape(1, HID), relu_out=True)
    z = _layer(Ar, h, d_col, W2, b2.reshape(1, OUT), relu_out=False)

    G = _gram(z)

    scores = _make_sc_decode()(G.reshape(-1), flat)
    return scores[:E]


# speedup vs baseline: 1.0238x; 1.0238x over previous
import functools

import jax
import jax.numpy as jnp
from jax import lax
from jax.experimental import pallas as pl
from jax.experimental.pallas import tpu as pltpu
from jax.experimental.pallas import tpu_sc as plsc

N = 4096
EMB = 128
HID = 128
OUT = 64
E = 1000000

NC = 2
NS = 16
NW = NC * NS
L = 16


_RPW = N // NW


@functools.cache
def _make_sc_prologue():
    @functools.partial(
        pl.kernel,
        mesh=plsc.VectorSubcoreMesh(core_axis_name="c", subcore_axis_name="s"),
        out_type=[
            jax.ShapeDtypeStruct((N, EMB), jnp.float32),
            jax.ShapeDtypeStruct((_EPAD,), jnp.int32),
        ],
        scratch_types=[
            pltpu.VMEM((_RPW,), jnp.int32),
            pltpu.VMEM((_RPW, EMB), jnp.float32),
            pltpu.VMEM((_EMX,), jnp.int32),
            pltpu.VMEM((_EMX,), jnp.int32),
            pltpu.SemaphoreType.DMA,
        ],
    )
    def _sc_prologue(emb_hbm, idx_hbm, src_hbm, dst_hbm, x_hbm, flat_hbm,
                     idx_v, rows_v, src_v, dst_v, sem):
        c = lax.axis_index("c")
        s = lax.axis_index("s")
        wid = s * NC + c
        rbase = wid * _RPW
        pltpu.sync_copy(idx_hbm.at[pl.ds(rbase, _RPW)], idx_v)
        pltpu.async_copy(emb_hbm.at[idx_v], rows_v, sem).wait()
        pltpu.sync_copy(rows_v, x_hbm.at[pl.ds(rbase, _RPW)])

        def prep(base, ew):
            pltpu.sync_copy(src_hbm.at[pl.ds(base, ew)], src_v.at[pl.ds(0, ew)])
            pltpu.sync_copy(dst_hbm.at[pl.ds(base, ew)], dst_v.at[pl.ds(0, ew)])

            def idx_body(g, carry):
                for l in range(8):
                    o = g * (8 * L) + l * L
                    sidx = src_v[pl.ds(o, L)]
                    t = dst_v[pl.ds(o, L)]
                    src_v[pl.ds(o, L)] = (
                        lax.shift_left(lax.shift_right_logical(t, 7), 19)
                        + lax.shift_left(sidx, 7)
                        + lax.bitwise_and(t, 127)
                    )
                return carry

            lax.fori_loop(0, ew // (8 * L), idx_body, 0)
            pltpu.sync_copy(src_v.at[pl.ds(0, ew)], flat_hbm.at[pl.ds(base, ew)])

        @pl.when(c == 0)
        def _():
            prep(s * _E0, _E0)

        @pl.when(c == 1)
        def _():
            prep(NS * _E0 + s * _E1, _E1)

    return _sc_prologue


_BI = 512
_BJ = 512


def _dnorm_body(a_ref, o_ref, ar_ref, acc_ref):
    i = pl.program_id(1)

    @pl.when(i == 0)
    def _():
        acc_ref[...] = jnp.zeros_like(acc_ref)

    a = jnp.maximum(a_ref[...], 0.0)
    ar_ref[...] = a.astype(jnp.bfloat16)
    acc_ref[...] += jnp.sum(a, axis=0, keepdims=True)

    @pl.when(i == pl.num_programs(1) - 1)
    def _():
        deg = acc_ref[...]
        o_ref[...] = jnp.where(deg > 0.0, lax.rsqrt(deg), 0.0)


def _dnorm(A):
    return pl.pallas_call(
        _dnorm_body,
        grid=(N // _BJ, N // _BI),
        in_specs=[pl.BlockSpec((_BI, _BJ), lambda j, i: (i, j))],
        out_specs=[
            pl.BlockSpec((1, _BJ), lambda j, i: (0, j)),
            pl.BlockSpec((_BI, _BJ), lambda j, i: (i, j)),
        ],
        out_shape=[
            jax.ShapeDtypeStruct((1, N), jnp.float32),
            jax.ShapeDtypeStruct((N, N), jnp.bfloat16),
        ],
        scratch_shapes=[pltpu.VMEM((1, _BJ), jnp.float32)],
    )(A)


def _layer_body(a_ref, x_ref, drow_ref, dcol_ref, w_ref, b_ref, o_ref,
                acc_ref, xls_ref, *, relu_out, bi):
    j = pl.program_id(0)
    i = pl.program_id(1)

    @pl.when(j == 0)
    def _():
        xw = jnp.dot(x_ref[pl.ds(i * bi, bi), :], w_ref[...],
                     preferred_element_type=jnp.float32)
        xls_ref[pl.ds(i * bi, bi), :] = (
            drow_ref[pl.ds(i * bi, bi), :] * xw).astype(jnp.bfloat16)

    @pl.when(i == 0)
    def _():
        acc_ref[...] = jnp.zeros_like(acc_ref)

    acc_ref[...] += lax.dot_general(
        a_ref[...], xls_ref[pl.ds(i * bi, bi), :],
        (((0,), (0,)), ((), ())),
        preferred_element_type=jnp.float32,
    )

    @pl.when(i == pl.num_programs(1) - 1)
    def _():
        y = dcol_ref[...] * acc_ref[...] + b_ref[...]
        o_ref[...] = jnp.maximum(y, 0.0) if relu_out else y


def _layer(Ar, x, d_col, W, b, relu_out):
    din = x.shape[1]
    dout = W.shape[1]
    body = functools.partial(_layer_body, relu_out=relu_out, bi=_BI)
    return pl.pallas_call(
        body,
        grid=(N // _BJ, N // _BI),
        in_specs=[
            pl.BlockSpec((_BI, _BJ), lambda j, i: (i, j)),
            pl.BlockSpec((N, din), lambda j, i: (0, 0)),
            pl.BlockSpec((N, 1), lambda j, i: (0, 0)),
            pl.BlockSpec((_BJ, 1), lambda j, i: (j, 0)),
            pl.BlockSpec((din, dout), lambda j, i: (0, 0)),
            pl.BlockSpec((1, dout), lambda j, i: (0, 0)),
        ],
        out_specs=pl.BlockSpec((_BJ, dout), lambda j, i: (j, 0)),
        out_shape=jax.ShapeDtypeStruct((N, dout), jnp.float32),
        scratch_shapes=[
            pltpu.VMEM((_BJ, dout), jnp.float32),
            pltpu.VMEM((N, dout), jnp.bfloat16),
        ],
    )(Ar, x, d_col, d_col, W, b)


_BGC = 512


def _gram_body(zi_ref, zc_ref, o_ref):
    res = lax.dot_general(
        zi_ref[...], zc_ref[...],
        (((1,), (1,)), ((), ())),
        preferred_element_type=jnp.float32,
    )
    for t in range(_BGC // 128):
        o_ref[t, :, :] = res[:, t * 128:(t + 1) * 128]


def _gram(z):
    return pl.pallas_call(
        _gram_body,
        grid=(N // _BGC,),
        in_specs=[
            pl.BlockSpec((N, OUT), lambda c: (0, 0)),
            pl.BlockSpec((_BGC, OUT), lambda c: (c, 0)),
        ],
        out_specs=pl.BlockSpec((_BGC // 128, N, 128), lambda c: (c, 0, 0)),
        out_shape=jax.ShapeDtypeStruct((N // 128, N, 128), jnp.float32),
    )(z, z)


_CH = 128
_KF = 8
_N0 = 392
_N1 = 104
_E0 = _N0 * _CH
_E1 = _N1 * _CH
_EMX = max(_E0, _E1)
_EPAD = NS * (_E0 + _E1)


@functools.cache
def _make_sc_decode():
    @functools.partial(
        pl.kernel,
        mesh=plsc.VectorSubcoreMesh(core_axis_name="c", subcore_axis_name="s"),
        out_type=jax.ShapeDtypeStruct((_EPAD,), jnp.float32),
        scratch_types=[
            pltpu.VMEM((_EMX,), jnp.int32),
            pltpu.VMEM((_EMX,), jnp.float32),
            pltpu.SemaphoreType.DMA,
        ],
    )
    def _sc_decode(g_hbm, flat_hbm, out_hbm, flat_v, res_v, sem):
        c = lax.axis_index("c")
        s = lax.axis_index("s")

        def run(base, nchunks):
            ew = nchunks * _CH
            pltpu.sync_copy(flat_hbm.at[pl.ds(base, ew)], flat_v.at[pl.ds(0, ew)])

            def group(g, carry):
                copies = []
                for j in range(_KF):
                    off = (g * _KF + j) * _CH
                    copies.append(
                        pltpu.async_copy(g_hbm.at[flat_v.at[pl.ds(off, _CH)]],
                                         res_v.at[pl.ds(off, _CH)], sem))
                for cp in copies:
                    cp.wait()
                return carry

            lax.fori_loop(0, nchunks // _KF, group, 0)
            pltpu.sync_copy(res_v.at[pl.ds(0, ew)], out_hbm.at[pl.ds(base, ew)])

        @pl.when(c == 0)
        def _():
            run(s * _E0, _N0)

        @pl.when(c == 1)
        def _():
            run(NS * _E0 + s * _E1, _N1)

    return _sc_decode


def kernel(node_indices, attn_matrix, edge_label_index, emb_table, W1, b1, W2, b2):
    idx32 = node_indices.astype(jnp.int32)
    eli = edge_label_index.astype(jnp.int32)
    pad = jnp.zeros((2, _EPAD - E), jnp.int32)
    eli_p = jnp.concatenate([eli, pad], axis=1)
    x, flat = _make_sc_prologue()(emb_table, idx32, eli_p[0], eli_p[1])

    d_row, Ar = _dnorm(attn_matrix)
    d_col = d_row.reshape(N, 1)

    h = _layer(Ar, x, d_col, W1, b1.reshape(1, HID), relu_out=True)
    z = _layer(Ar, h, d_col, W2, b2.reshape(1, OUT), relu_out=False)

    G = _gram(z)

    scores = _make_sc_decode()(G.reshape(-1), flat)
    return scores[:E]
